# baseline (speedup 1.0000x reference)
import functools

import jax
import jax.numpy as jnp
import numpy as np
from jax import lax
from jax.experimental import pallas as pl
from jax.experimental.pallas import tpu as pltpu

R = 64
C = 1_000_000
W = 8192
CHUNK = 128
NB = (C + W - 1) // W

_MINVAL = np.float32(1e-7)
_MAXVAL = np.float32(1.0 - 1e-7)
_SCALE = np.float32(_MAXVAL - _MINVAL)
_KS0 = 0
_KS1 = 42
_KS2 = 42 ^ 0x1BD11BDA
_ROT = (13, 15, 26, 6, 17, 29, 16, 24)


def _rotl(x, d):
    return lax.shift_left(x, jnp.int32(d)) | lax.shift_right_logical(
        x, jnp.int32(32 - d))


def _threefry_bits(flat):
    ks = (jnp.int32(_KS0), jnp.int32(_KS1), jnp.int32(_KS2))
    x0 = jnp.zeros_like(flat) + ks[0]
    x1 = flat + ks[1]
    for g in range(5):
        rots = _ROT[0:4] if g % 2 == 0 else _ROT[4:8]
        for d in rots:
            x0 = x0 + x1
            x1 = _rotl(x1, d)
            x1 = x1 ^ x0
        x0 = x0 + ks[(g + 1) % 3]
        x1 = x1 + ks[(g + 2) % 3] + jnp.int32(g + 1)
    return x0 ^ x1


def _gumbel_from_flat(flat):
    bits = _threefry_bits(flat)
    fb = lax.shift_right_logical(bits, jnp.int32(9)) | jnp.int32(0x3F800000)
    fl = lax.bitcast_convert_type(fb, jnp.float32) - jnp.float32(1.0)
    u = jnp.maximum(_MINVAL, fl * _SCALE + _MINVAL)
    return -jnp.log(-jnp.log(u))


def _kernel(x_ref, o_ref, rm_ref, ri_ref):
    j = pl.program_id(0)

    @pl.when(j == 0)
    def _init():
        rm_ref[:] = jnp.full((R, 1), -jnp.inf, jnp.float32)
        ri_ref[:] = jnp.zeros((R, 1), jnp.int32)

    base = j * W
    row = lax.broadcasted_iota(jnp.int32, (R, CHUNK), 0)
    lane = lax.broadcasted_iota(jnp.int32, (R, CHUNK), 1)

    def body(t, carry):
        rm, ri = carry
        off = pl.multiple_of(t * CHUNK, CHUNK)
        col = base + off + lane
        flat = row * C + col
        g = _gumbel_from_flat(flat)
        v = x_ref[:, pl.ds(off, CHUNK)] + g
        v = jnp.where(col < C, v, -jnp.inf)
        take = v > rm
        rm = jnp.where(take, v, rm)
        ri = jnp.where(take, col, ri)
        return rm, ri

    rm0 = jnp.full((R, CHUNK), -jnp.inf, jnp.float32)
    ri0 = jnp.zeros((R, CHUNK), jnp.int32)
    rm, ri = lax.fori_loop(0, W // CHUNK, body, (rm0, ri0))

    bm = jnp.max(rm, axis=1, keepdims=True)
    bi = jnp.min(jnp.where(rm == bm, ri, jnp.int32(0x7FFFFFFF)),
                 axis=1, keepdims=True)

    take = bm > rm_ref[:]
    rm_ref[:] = jnp.where(take, bm, rm_ref[:])
    ri_ref[:] = jnp.where(take, bi, ri_ref[:])

    @pl.when(j == NB - 1)
    def _fin():
        o_ref[:] = ri_ref[:]


@jax.jit
def kernel(logits):
    out = pl.pallas_call(
        _kernel,
        grid=(NB,),
        in_specs=[pl.BlockSpec((R, W), lambda j: (0, j))],
        out_specs=pl.BlockSpec((R, 1), lambda j: (0, 0)),
        out_shape=jax.ShapeDtypeStruct((R, 1), jnp.int32),
        scratch_shapes=[
            pltpu.VMEM((R, 1), jnp.float32),
            pltpu.VMEM((R, 1), jnp.int32),
        ],
    )(logits)
    return out.reshape(R)
